# baseline (speedup 1.0000x reference)
import functools

import jax
import jax.numpy as jnp
from jax import lax
from jax.experimental import pallas as pl
from jax.experimental.pallas import tpu as pltpu
from jax.experimental.pallas import tpu_sc as plsc

N_NODES = 10000
N_EDGES = 160000
D_IN = 256
D_OUT = 512

H = 128
NC = 2
NS = 16
B = 80
EPT = N_EDGES // NS
NB = EPT // B
AROWS = N_NODES + 8
K = 3
IK = 6
NZT = 10
SLAB = N_NODES // NZT


def _sc_aggregate(xg, e4, zrows, zcol, ones1):
    mesh = plsc.VectorSubcoreMesh(
        core_axis_name="c", subcore_axis_name="s", num_cores=NC,
        num_subcores=NS)

    @functools.partial(
        pl.kernel,
        out_type=(jax.ShapeDtypeStruct((NC * N_NODES, H), jnp.float32),
                  jax.ShapeDtypeStruct((N_NODES, 16), jnp.float32)),
        mesh=mesh,
        scratch_types=[
            pltpu.VMEM_SHARED((AROWS, H), jnp.float32),
            pltpu.VMEM_SHARED((AROWS, 16), jnp.float32),
            pltpu.VMEM((IK, B), jnp.int32),
            pltpu.VMEM((IK, B), jnp.int32),
            pltpu.VMEM((K, B), jnp.int32),
            pltpu.VMEM((K, B, H), jnp.float32),
            pltpu.VMEM((B, 16), jnp.float32),
            pltpu.SemaphoreType.DMA((IK,)),
            pltpu.SemaphoreType.DMA((K,)),
            pltpu.SemaphoreType.DMA((K,)),
            pltpu.SemaphoreType.DMA,
        ],
        compiler_params=pltpu.CompilerParams(use_tc_tiling_on_sc=False),
    )
    def agg(xg_hbm, e_hbm, z_hbm, zc_hbm, ones_hbm, out_hbm, cnt_hbm,
            acc, cac, row_r, col_r, gix_r, feat_r, ones_v,
            isem, gsem, ssem, csem):
        c = lax.axis_index("c")
        s = lax.axis_index("s")

        def start_idx(b, p):
            pltpu.async_copy(e_hbm.at[0].at[s].at[b], row_r.at[p],
                             isem.at[p])
            pltpu.async_copy(e_hbm.at[1].at[s].at[b], col_r.at[p],
                             isem.at[p])

        def wait_idx(p):
            pltpu.make_async_copy(e_hbm.at[0].at[s].at[0], row_r.at[p],
                                  isem.at[p]).wait()
            pltpu.make_async_copy(e_hbm.at[0].at[s].at[0], col_r.at[p],
                                  isem.at[p]).wait()

        def start_gather(p, ip):
            for j in range(B // 16):
                c16 = col_r[ip, pl.ds(j * 16, 16)]
                gix_r[p, pl.ds(j * 16, 16)] = c16 * 2 + c
            pltpu.async_copy(xg_hbm.at[gix_r.at[p]], feat_r.at[p],
                             gsem.at[p])

        def wait_gather(p):
            pltpu.make_async_copy(xg_hbm.at[gix_r.at[p]], feat_r.at[p],
                                  gsem.at[p]).wait()

        def wait_scatter(p):
            pltpu.make_async_copy(feat_r.at[p], acc.at[row_r.at[0]],
                                  ssem.at[p]).wait()

        for p in range(IK):
            start_idx(p, p)
        pltpu.sync_copy(ones_hbm, ones_v)

        @pl.when(s < NZT)
        def _():
            pltpu.sync_copy(z_hbm, acc.at[pl.ds(s * SLAB, SLAB)])
            pltpu.sync_copy(zc_hbm, cac.at[pl.ds(s * SLAB, SLAB)])
        plsc.subcore_barrier()
        for t in range(2):
            wait_idx(t)
            start_gather(t, t)

        def body(b, carry):
            p = lax.rem(b, K)
            ip = lax.rem(b, IK)

            @pl.when(b + 2 < NB)
            def _():
                iq = lax.rem(b + 2, IK)
                wait_idx(iq)
                start_gather(lax.rem(b + 2, K), iq)
            wait_gather(p)
            pltpu.async_copy(feat_r.at[p], acc.at[row_r.at[ip]],
                             ssem.at[p], add=True)
            pltpu.async_copy(ones_v, cac.at[row_r.at[ip]], csem, add=True)

            @pl.when(b + K < NB)
            def _():
                wait_scatter(p)

                @pl.when(b + IK < NB)
                def _():
                    start_idx(b + IK, ip)
            return carry

        lax.fori_loop(0, NB, body, 0)
        for p in range(K):
            wait_scatter(p)

        def drain(b, carry):
            pltpu.make_async_copy(ones_v, cac.at[row_r.at[0]], csem).wait()
            return carry

        lax.fori_loop(0, NB, drain, 0)
        plsc.subcore_barrier()

        @pl.when(s < NZT)
        def _():
            pltpu.sync_copy(acc.at[pl.ds(s * SLAB, SLAB)],
                            out_hbm.at[pl.ds(c * N_NODES + s * SLAB, SLAB)])

        @pl.when((s < NZT) & (c == 0))
        def _():
            pltpu.sync_copy(cac.at[pl.ds(s * SLAB, SLAB)],
                            cnt_hbm.at[pl.ds(s * SLAB, SLAB)])

    return agg(xg, e4, zrows, zcol, ones1)


def _tc_self(x, W_self, b_self2, b_nei2):
    R = 1000
    NR = N_NODES // R

    def body(x_ref, ws_ref, bs_ref, bn_ref, o_ref):
        dn = (((1,), (1,)), ((), ()))
        h = lax.dot_general(x_ref[...], ws_ref[...], dn,
                            preferred_element_type=jnp.float32)
        o_ref[...] = h + bs_ref[...] + bn_ref[...]

    return pl.pallas_call(
        body,
        grid=(NR,),
        in_specs=[
            pl.BlockSpec((R, D_IN), lambda i: (i, 0)),
            pl.BlockSpec((D_OUT, D_IN), lambda i: (0, 0)),
            pl.BlockSpec((1, D_OUT), lambda i: (0, 0)),
            pl.BlockSpec((1, D_OUT), lambda i: (0, 0)),
        ],
        out_specs=pl.BlockSpec((R, D_OUT), lambda i: (i, 0)),
        out_shape=jax.ShapeDtypeStruct((N_NODES, D_OUT), jnp.float32),
    )(x, W_self, b_self2, b_nei2)


def _tc_combine(h_self, feat, cnt, W_nei):
    R = 1000
    NR = N_NODES // R

    def body(hs_ref, lo_ref, hi_ref, cnt_ref, wn_ref, o_ref):
        inv = 1.0 / (cnt_ref[...][:, :1] + 1e-12)
        nlo = lo_ref[...] * inv
        nhi = hi_ref[...] * inv
        wn = wn_ref[...]
        dn = (((1,), (1,)), ((), ()))
        h = hs_ref[...] + lax.dot_general(nlo, wn[:, :H], dn,
                                          preferred_element_type=jnp.float32)
        h = h + lax.dot_general(nhi, wn[:, H:], dn,
                                preferred_element_type=jnp.float32)
        o_ref[...] = jnp.maximum(h, 0.0)

    return pl.pallas_call(
        body,
        grid=(NR,),
        in_specs=[
            pl.BlockSpec((R, D_OUT), lambda i: (i, 0)),
            pl.BlockSpec((R, H), lambda i: (i, 0)),
            pl.BlockSpec((R, H), lambda i: (NR + i, 0)),
            pl.BlockSpec((R, 16), lambda i: (i, 0)),
            pl.BlockSpec((D_OUT, D_IN), lambda i: (0, 0)),
        ],
        out_specs=pl.BlockSpec((R, D_OUT), lambda i: (i, 0)),
        out_shape=jax.ShapeDtypeStruct((N_NODES, D_OUT), jnp.float32),
    )(h_self, feat, feat, cnt, W_nei)


def kernel(x, edge_index, W_self, b_self, W_nei, b_nei):
    e4 = edge_index.astype(jnp.int32).reshape(2, NS, NB, B)
    xg = x.reshape(2 * N_NODES, H)
    zrows = jnp.zeros((SLAB, H), jnp.float32)
    zcol = jnp.zeros((SLAB, 16), jnp.float32)
    ones1 = jnp.ones((B, 16), jnp.float32)
    feat, cnt = _sc_aggregate(xg, e4, zrows, zcol, ones1)
    h_self = _tc_self(x, W_self, b_self.reshape(1, D_OUT),
                      b_nei.reshape(1, D_OUT))
    return _tc_combine(h_self, feat, cnt, W_nei)


# speedup vs baseline: 1.0275x; 1.0275x over previous
import functools

import jax
import jax.numpy as jnp
from jax import lax
from jax.experimental import pallas as pl
from jax.experimental.pallas import tpu as pltpu
from jax.experimental.pallas import tpu_sc as plsc

N_NODES = 10000
N_EDGES = 160000
D_IN = 256
D_OUT = 512

H = 128
NC = 2
NS = 16
B = 80
EPT = N_EDGES // NS
NB = EPT // B
AROWS = N_NODES + 8
K = 3
IK = 6
NZT = 10
SLAB = N_NODES // NZT


def _sc_aggregate(xg, ef, zrows, zcol, ones1):
    mesh = plsc.VectorSubcoreMesh(
        core_axis_name="c", subcore_axis_name="s", num_cores=NC,
        num_subcores=NS)

    @functools.partial(
        pl.kernel,
        out_type=(jax.ShapeDtypeStruct((NC * N_NODES, H), jnp.float32),
                  jax.ShapeDtypeStruct((N_NODES, 16), jnp.float32)),
        mesh=mesh,
        scratch_types=[
            pltpu.VMEM_SHARED((AROWS, H), jnp.float32),
            pltpu.VMEM_SHARED((AROWS, 16), jnp.float32),
            pltpu.VMEM((IK, B), jnp.int32),
            pltpu.VMEM((IK, B), jnp.int32),
            pltpu.VMEM((K, B), jnp.int32),
            pltpu.VMEM((K, B, H), jnp.float32),
            pltpu.VMEM((B, 16), jnp.float32),
            pltpu.SemaphoreType.DMA((IK,)),
            pltpu.SemaphoreType.DMA((K,)),
            pltpu.SemaphoreType.DMA((K,)),
            pltpu.SemaphoreType.DMA,
        ],
        compiler_params=pltpu.CompilerParams(use_tc_tiling_on_sc=False),
    )
    def agg(xg_hbm, e_hbm, z_hbm, zc_hbm, ones_hbm, out_hbm, cnt_hbm,
            acc, cac, row_r, col_r, gix_r, feat_r, ones_v,
            isem, gsem, ssem, csem):
        c = lax.axis_index("c")
        s = lax.axis_index("s")

        def start_idx(b, p):
            off = s * EPT + b * B
            pltpu.async_copy(e_hbm.at[pl.ds(off, B)], row_r.at[p],
                             isem.at[p])
            pltpu.async_copy(e_hbm.at[pl.ds(N_EDGES + off, B)],
                             col_r.at[p], isem.at[p])

        def wait_idx(p):
            pltpu.make_async_copy(e_hbm.at[pl.ds(0, B)], row_r.at[p],
                                  isem.at[p]).wait()
            pltpu.make_async_copy(e_hbm.at[pl.ds(0, B)], col_r.at[p],
                                  isem.at[p]).wait()

        def start_gather(p, ip):
            for j in range(B // 16):
                c16 = col_r[ip, pl.ds(j * 16, 16)]
                gix_r[p, pl.ds(j * 16, 16)] = c16 * 2 + c
            pltpu.async_copy(xg_hbm.at[gix_r.at[p]], feat_r.at[p],
                             gsem.at[p])

        def wait_gather(p):
            pltpu.make_async_copy(xg_hbm.at[gix_r.at[p]], feat_r.at[p],
                                  gsem.at[p]).wait()

        def wait_scatter(p):
            pltpu.make_async_copy(feat_r.at[p], acc.at[row_r.at[0]],
                                  ssem.at[p]).wait()

        for p in range(IK):
            start_idx(p, p)
        pltpu.sync_copy(ones_hbm, ones_v)

        @pl.when(s < NZT)
        def _():
            pltpu.sync_copy(z_hbm, acc.at[pl.ds(s * SLAB, SLAB)])
            pltpu.sync_copy(zc_hbm, cac.at[pl.ds(s * SLAB, SLAB)])
        plsc.subcore_barrier()
        for t in range(2):
            wait_idx(t)
            start_gather(t, t)

        def body(b, carry):
            p = lax.rem(b, K)
            ip = lax.rem(b, IK)

            @pl.when(b + 2 < NB)
            def _():
                iq = lax.rem(b + 2, IK)
                wait_idx(iq)
                start_gather(lax.rem(b + 2, K), iq)
            wait_gather(p)
            pltpu.async_copy(feat_r.at[p], acc.at[row_r.at[ip]],
                             ssem.at[p], add=True)
            pltpu.async_copy(ones_v, cac.at[row_r.at[ip]], csem, add=True)

            @pl.when(b + K < NB)
            def _():
                wait_scatter(p)

                @pl.when(b + IK < NB)
                def _():
                    start_idx(b + IK, ip)
            return carry

        lax.fori_loop(0, NB, body, 0)
        for p in range(K):
            wait_scatter(p)

        def drain(b, carry):
            pltpu.make_async_copy(ones_v, cac.at[row_r.at[0]], csem).wait()
            return carry

        lax.fori_loop(0, NB, drain, 0)
        plsc.subcore_barrier()

        @pl.when(s < NZT)
        def _():
            pltpu.sync_copy(acc.at[pl.ds(s * SLAB, SLAB)],
                            out_hbm.at[pl.ds(c * N_NODES + s * SLAB, SLAB)])

        @pl.when((s < NZT) & (c == 0))
        def _():
            pltpu.sync_copy(cac.at[pl.ds(s * SLAB, SLAB)],
                            cnt_hbm.at[pl.ds(s * SLAB, SLAB)])

    return agg(xg, ef, zrows, zcol, ones1)


def _tc_combine(x, feat, cnt, W_self, W_nei, b_self2, b_nei2):
    R = 1000
    NR = N_NODES // R

    def body(x_ref, lo_ref, hi_ref, cnt_ref, ws_ref, wn_ref, bs_ref,
             bn_ref, o_ref):
        inv = 1.0 / (cnt_ref[...][:, :1] + 1e-12)
        nlo = (lo_ref[...] * inv).astype(jnp.bfloat16)
        nhi = (hi_ref[...] * inv).astype(jnp.bfloat16)
        xb = x_ref[...].astype(jnp.bfloat16)
        ws = ws_ref[...].astype(jnp.bfloat16)
        wn = wn_ref[...].astype(jnp.bfloat16)
        dn = (((1,), (1,)), ((), ()))
        h = lax.dot_general(xb, ws, dn,
                            preferred_element_type=jnp.float32)
        h = h + lax.dot_general(nlo, wn[:, :H], dn,
                                preferred_element_type=jnp.float32)
        h = h + lax.dot_general(nhi, wn[:, H:], dn,
                                preferred_element_type=jnp.float32)
        h = h + bs_ref[...] + bn_ref[...]
        o_ref[...] = jnp.maximum(h, 0.0)

    return pl.pallas_call(
        body,
        grid=(NR,),
        in_specs=[
            pl.BlockSpec((R, D_IN), lambda i: (i, 0)),
            pl.BlockSpec((R, H), lambda i: (i, 0)),
            pl.BlockSpec((R, H), lambda i: (NR + i, 0)),
            pl.BlockSpec((R, 16), lambda i: (i, 0)),
            pl.BlockSpec((D_OUT, D_IN), lambda i: (0, 0)),
            pl.BlockSpec((D_OUT, D_IN), lambda i: (0, 0)),
            pl.BlockSpec((1, D_OUT), lambda i: (0, 0)),
            pl.BlockSpec((1, D_OUT), lambda i: (0, 0)),
        ],
        out_specs=pl.BlockSpec((R, D_OUT), lambda i: (i, 0)),
        out_shape=jax.ShapeDtypeStruct((N_NODES, D_OUT), jnp.float32),
    )(x, feat, feat, cnt, W_self, W_nei, b_self2, b_nei2)


def kernel(x, edge_index, W_self, b_self, W_nei, b_nei):
    ef = edge_index.astype(jnp.int32).reshape(2 * N_EDGES)
    xg = x.reshape(2 * N_NODES, H)
    zrows = jnp.zeros((SLAB, H), jnp.float32)
    zcol = jnp.zeros((SLAB, 16), jnp.float32)
    ones1 = jnp.ones((B, 16), jnp.float32)
    feat, cnt = _sc_aggregate(xg, ef, zrows, zcol, ones1)
    return _tc_combine(x, feat, cnt, W_self, W_nei,
                       b_self.reshape(1, D_OUT), b_nei.reshape(1, D_OUT))


# speedup vs baseline: 1.0318x; 1.0043x over previous
import functools

import jax
import jax.numpy as jnp
from jax import lax
from jax.experimental import pallas as pl
from jax.experimental.pallas import tpu as pltpu
from jax.experimental.pallas import tpu_sc as plsc

N_NODES = 10000
N_EDGES = 160000
D_IN = 256
D_OUT = 512

H = 128
NC = 2
NS = 16
B = 80
EPT = N_EDGES // NS
NB = EPT // B
AROWS = N_NODES + 8
K = 3
IK = 6
NZT = 10
SLAB = N_NODES // NZT


def _sc_aggregate(xg, ef, zrows, zcol, ones1):
    mesh = plsc.VectorSubcoreMesh(
        core_axis_name="c", subcore_axis_name="s", num_cores=NC,
        num_subcores=NS)

    @functools.partial(
        pl.kernel,
        out_type=(jax.ShapeDtypeStruct((NC * N_NODES, H), jnp.float32),
                  jax.ShapeDtypeStruct((N_NODES, 16), jnp.float32)),
        mesh=mesh,
        scratch_types=[
            pltpu.VMEM_SHARED((AROWS, H), jnp.float32),
            pltpu.VMEM_SHARED((AROWS, 16), jnp.float32),
            pltpu.VMEM((IK, B), jnp.int32),
            pltpu.VMEM((IK, B), jnp.int32),
            pltpu.VMEM((K, B), jnp.int32),
            pltpu.VMEM((K, B, H), jnp.float32),
            pltpu.VMEM((B, 16), jnp.float32),
            pltpu.SemaphoreType.DMA((IK,)),
            pltpu.SemaphoreType.DMA((K,)),
            pltpu.SemaphoreType.DMA((K,)),
            pltpu.SemaphoreType.DMA,
        ],
        compiler_params=pltpu.CompilerParams(use_tc_tiling_on_sc=False),
    )
    def agg(xg_hbm, e_hbm, z_hbm, zc_hbm, ones_hbm, out_hbm, cnt_hbm,
            acc, cac, row_r, col_r, gix_r, feat_r, ones_v,
            isem, gsem, ssem, csem):
        c = lax.axis_index("c")
        s = lax.axis_index("s")

        def start_idx(b, p):
            off = s * EPT + b * B
            pltpu.async_copy(e_hbm.at[pl.ds(off, B)], row_r.at[p],
                             isem.at[p])
            pltpu.async_copy(e_hbm.at[pl.ds(N_EDGES + off, B)],
                             col_r.at[p], isem.at[p])

        def wait_idx(p):
            pltpu.make_async_copy(e_hbm.at[pl.ds(0, B)], row_r.at[p],
                                  isem.at[p]).wait()
            pltpu.make_async_copy(e_hbm.at[pl.ds(0, B)], col_r.at[p],
                                  isem.at[p]).wait()

        def start_gather(p, ip):
            for j in range(B // 16):
                c16 = col_r[ip, pl.ds(j * 16, 16)]
                gix_r[p, pl.ds(j * 16, 16)] = c16 * 2 + c
            pltpu.async_copy(xg_hbm.at[gix_r.at[p]], feat_r.at[p],
                             gsem.at[p])

        def wait_gather(p):
            pltpu.make_async_copy(xg_hbm.at[gix_r.at[p]], feat_r.at[p],
                                  gsem.at[p]).wait()

        def wait_scatter(p):
            pltpu.make_async_copy(feat_r.at[p], acc.at[row_r.at[0]],
                                  ssem.at[p]).wait()

        for p in range(IK):
            start_idx(p, p)
        pltpu.sync_copy(ones_hbm, ones_v)

        @pl.when(s < NZT)
        def _():
            pltpu.sync_copy(z_hbm, acc.at[pl.ds(s * SLAB, SLAB)])
            pltpu.sync_copy(zc_hbm, cac.at[pl.ds(s * SLAB, SLAB)])
        plsc.subcore_barrier()
        for t in range(2):
            wait_idx(t)
            start_gather(t, t)

        def body(b, carry):
            p = lax.rem(b, K)
            ip = lax.rem(b, IK)

            @pl.when(b + 2 < NB)
            def _():
                iq = lax.rem(b + 2, IK)
                wait_idx(iq)
                start_gather(lax.rem(b + 2, K), iq)
            wait_gather(p)
            pltpu.async_copy(feat_r.at[p], acc.at[row_r.at[ip]],
                             ssem.at[p], add=True)
            pltpu.async_copy(ones_v, cac.at[row_r.at[ip]], csem, add=True)

            @pl.when(b + K < NB)
            def _():
                wait_scatter(p)

                @pl.when(b + IK < NB)
                def _():
                    start_idx(b + IK, ip)
            return carry

        lax.fori_loop(0, NB, body, 0)
        for p in range(K):
            wait_scatter(p)

        def drain(b, carry):
            pltpu.make_async_copy(ones_v, cac.at[row_r.at[0]], csem).wait()
            return carry

        lax.fori_loop(0, NB, drain, 0)
        plsc.subcore_barrier()

        @pl.when(s < NZT)
        def _():
            pltpu.sync_copy(acc.at[pl.ds(s * SLAB, SLAB)],
                            out_hbm.at[pl.ds(c * N_NODES + s * SLAB, SLAB)])

        @pl.when((s < NZT) & (c == 0))
        def _():
            pltpu.sync_copy(cac.at[pl.ds(s * SLAB, SLAB)],
                            cnt_hbm.at[pl.ds(s * SLAB, SLAB)])

    return agg(xg, ef, zrows, zcol, ones1)


def _tc_combine(x, feat, cnt, W_self, W_nei, b_self2, b_nei2):
    R = 2000
    NR = N_NODES // R

    def body(x_ref, lo_ref, hi_ref, cnt_ref, ws_ref, wn_ref, bs_ref,
             bn_ref, o_ref):
        inv = 1.0 / (cnt_ref[...][:, :1] + 1e-12)
        nlo = lo_ref[...] * inv
        nhi = hi_ref[...] * inv
        xb = x_ref[...]
        ws = ws_ref[...]
        wn = wn_ref[...]
        dn = (((1,), (1,)), ((), ()))
        h = lax.dot_general(xb, ws, dn,
                            preferred_element_type=jnp.float32)
        h = h + lax.dot_general(nlo, wn[:, :H], dn,
                                preferred_element_type=jnp.float32)
        h = h + lax.dot_general(nhi, wn[:, H:], dn,
                                preferred_element_type=jnp.float32)
        h = h + bs_ref[...] + bn_ref[...]
        o_ref[...] = jnp.maximum(h, 0.0)

    return pl.pallas_call(
        body,
        grid=(NR,),
        in_specs=[
            pl.BlockSpec((R, D_IN), lambda i: (i, 0)),
            pl.BlockSpec((R, H), lambda i: (i, 0)),
            pl.BlockSpec((R, H), lambda i: (NR + i, 0)),
            pl.BlockSpec((R, 16), lambda i: (i, 0)),
            pl.BlockSpec((D_OUT, D_IN), lambda i: (0, 0)),
            pl.BlockSpec((D_OUT, D_IN), lambda i: (0, 0)),
            pl.BlockSpec((1, D_OUT), lambda i: (0, 0)),
            pl.BlockSpec((1, D_OUT), lambda i: (0, 0)),
        ],
        out_specs=pl.BlockSpec((R, D_OUT), lambda i: (i, 0)),
        out_shape=jax.ShapeDtypeStruct((N_NODES, D_OUT), jnp.float32),
    )(x, feat, feat, cnt, W_self, W_nei, b_self2, b_nei2)


def kernel(x, edge_index, W_self, b_self, W_nei, b_nei):
    ef = edge_index.astype(jnp.int32).reshape(2 * N_EDGES)
    xg = x.reshape(2 * N_NODES, H)
    zrows = jnp.zeros((SLAB, H), jnp.float32)
    zcol = jnp.zeros((SLAB, 16), jnp.float32)
    ones1 = jnp.ones((B, 16), jnp.float32)
    feat, cnt = _sc_aggregate(xg, ef, zrows, zcol, ones1)
    return _tc_combine(x, feat, cnt, W_self, W_nei,
                       b_self.reshape(1, D_OUT), b_nei.reshape(1, D_OUT))


# speedup vs baseline: 1.0388x; 1.0067x over previous
import functools

import jax
import jax.numpy as jnp
from jax import lax
from jax.experimental import pallas as pl
from jax.experimental.pallas import tpu as pltpu
from jax.experimental.pallas import tpu_sc as plsc

N_NODES = 10000
N_EDGES = 160000
D_IN = 256
D_OUT = 512

H = 128
NC = 2
NS = 16
B = 80
EPT = N_EDGES // NS
NB = EPT // B
AROWS = N_NODES + 8
K = 3
IK = 6
NZT = 16
SLAB = N_NODES // NZT


def _sc_aggregate(xg, ef, zrows, zcol, ones1):
    mesh = plsc.VectorSubcoreMesh(
        core_axis_name="c", subcore_axis_name="s", num_cores=NC,
        num_subcores=NS)

    @functools.partial(
        pl.kernel,
        out_type=(jax.ShapeDtypeStruct((NC * N_NODES, H), jnp.float32),
                  jax.ShapeDtypeStruct((N_NODES, 16), jnp.float32)),
        mesh=mesh,
        scratch_types=[
            pltpu.VMEM_SHARED((AROWS, H), jnp.float32),
            pltpu.VMEM_SHARED((AROWS, 16), jnp.float32),
            pltpu.VMEM((IK, B), jnp.int32),
            pltpu.VMEM((IK, B), jnp.int32),
            pltpu.VMEM((K, B), jnp.int32),
            pltpu.VMEM((K, B, H), jnp.float32),
            pltpu.VMEM((B, 16), jnp.float32),
            pltpu.SemaphoreType.DMA((IK,)),
            pltpu.SemaphoreType.DMA((K,)),
            pltpu.SemaphoreType.DMA((K,)),
            pltpu.SemaphoreType.DMA,
        ],
        compiler_params=pltpu.CompilerParams(use_tc_tiling_on_sc=False),
    )
    def agg(xg_hbm, e_hbm, z_hbm, zc_hbm, ones_hbm, out_hbm, cnt_hbm,
            acc, cac, row_r, col_r, gix_r, feat_r, ones_v,
            isem, gsem, ssem, csem):
        c = lax.axis_index("c")
        s = lax.axis_index("s")

        def start_idx(b, p):
            off = s * EPT + b * B
            pltpu.async_copy(e_hbm.at[pl.ds(off, B)], row_r.at[p],
                             isem.at[p])
            pltpu.async_copy(e_hbm.at[pl.ds(N_EDGES + off, B)],
                             col_r.at[p], isem.at[p])

        def wait_idx(p):
            pltpu.make_async_copy(e_hbm.at[pl.ds(0, B)], row_r.at[p],
                                  isem.at[p]).wait()
            pltpu.make_async_copy(e_hbm.at[pl.ds(0, B)], col_r.at[p],
                                  isem.at[p]).wait()

        def start_gather(p, ip):
            for j in range(B // 16):
                c16 = col_r[ip, pl.ds(j * 16, 16)]
                gix_r[p, pl.ds(j * 16, 16)] = c16 * 2 + c
            pltpu.async_copy(xg_hbm.at[gix_r.at[p]], feat_r.at[p],
                             gsem.at[p])

        def wait_gather(p):
            pltpu.make_async_copy(xg_hbm.at[gix_r.at[p]], feat_r.at[p],
                                  gsem.at[p]).wait()

        def wait_scatter(p):
            pltpu.make_async_copy(feat_r.at[p], acc.at[row_r.at[0]],
                                  ssem.at[p]).wait()

        for p in range(IK):
            start_idx(p, p)
        pltpu.sync_copy(ones_hbm, ones_v)

        @pl.when(s < NZT)
        def _():
            pltpu.sync_copy(z_hbm, acc.at[pl.ds(s * SLAB, SLAB)])
            pltpu.sync_copy(zc_hbm, cac.at[pl.ds(s * SLAB, SLAB)])
        plsc.subcore_barrier()
        for t in range(2):
            wait_idx(t)
            start_gather(t, t)

        def body(b, carry):
            p = lax.rem(b, K)
            ip = lax.rem(b, IK)

            @pl.when(b >= 1)
            def _():
                wait_scatter(lax.rem(b - 1, K))

                @pl.when(b - 1 + IK < NB)
                def _():
                    start_idx(b - 1 + IK, lax.rem(b - 1, IK))

            @pl.when(b + 2 < NB)
            def _():
                iq = lax.rem(b + 2, IK)
                wait_idx(iq)
                start_gather(lax.rem(b + 2, K), iq)
            wait_gather(p)
            pltpu.async_copy(feat_r.at[p], acc.at[row_r.at[ip]],
                             ssem.at[p], add=True)
            pltpu.async_copy(ones_v, cac.at[row_r.at[ip]], csem, add=True)
            return carry

        lax.fori_loop(0, NB, body, 0)
        wait_scatter(lax.rem(NB - 1, K))

        def drain(b, carry):
            pltpu.make_async_copy(ones_v, cac.at[row_r.at[0]], csem).wait()
            return carry

        lax.fori_loop(0, NB, drain, 0)
        plsc.subcore_barrier()

        @pl.when(s < NZT)
        def _():
            pltpu.sync_copy(acc.at[pl.ds(s * SLAB, SLAB)],
                            out_hbm.at[pl.ds(c * N_NODES + s * SLAB, SLAB)])

        @pl.when((s < NZT) & (c == 0))
        def _():
            pltpu.sync_copy(cac.at[pl.ds(s * SLAB, SLAB)],
                            cnt_hbm.at[pl.ds(s * SLAB, SLAB)])

    return agg(xg, ef, zrows, zcol, ones1)


def _tc_combine(x, feat, cnt, W_self, W_nei, b_self2, b_nei2):
    R = 2000
    NR = N_NODES // R

    def body(x_ref, lo_ref, hi_ref, cnt_ref, ws_ref, wn_ref, bs_ref,
             bn_ref, o_ref):
        inv = 1.0 / (cnt_ref[...][:, :1] + 1e-12)
        nlo = lo_ref[...] * inv
        nhi = hi_ref[...] * inv
        xb = x_ref[...]
        ws = ws_ref[...]
        wn = wn_ref[...]
        dn = (((1,), (1,)), ((), ()))
        h = lax.dot_general(xb, ws, dn,
                            preferred_element_type=jnp.float32)
        h = h + lax.dot_general(nlo, wn[:, :H], dn,
                                preferred_element_type=jnp.float32)
        h = h + lax.dot_general(nhi, wn[:, H:], dn,
                                preferred_element_type=jnp.float32)
        h = h + bs_ref[...] + bn_ref[...]
        o_ref[...] = jnp.maximum(h, 0.0)

    return pl.pallas_call(
        body,
        grid=(NR,),
        in_specs=[
            pl.BlockSpec((R, D_IN), lambda i: (i, 0)),
            pl.BlockSpec((R, H), lambda i: (i, 0)),
            pl.BlockSpec((R, H), lambda i: (NR + i, 0)),
            pl.BlockSpec((R, 16), lambda i: (i, 0)),
            pl.BlockSpec((D_OUT, D_IN), lambda i: (0, 0)),
            pl.BlockSpec((D_OUT, D_IN), lambda i: (0, 0)),
            pl.BlockSpec((1, D_OUT), lambda i: (0, 0)),
            pl.BlockSpec((1, D_OUT), lambda i: (0, 0)),
        ],
        out_specs=pl.BlockSpec((R, D_OUT), lambda i: (i, 0)),
        out_shape=jax.ShapeDtypeStruct((N_NODES, D_OUT), jnp.float32),
    )(x, feat, feat, cnt, W_self, W_nei, b_self2, b_nei2)


def kernel(x, edge_index, W_self, b_self, W_nei, b_nei):
    ef = edge_index.astype(jnp.int32).reshape(2 * N_EDGES)
    xg = x.reshape(2 * N_NODES, H)
    zrows = jnp.zeros((SLAB, H), jnp.float32)
    zcol = jnp.zeros((SLAB, 16), jnp.float32)
    ones1 = jnp.ones((B, 16), jnp.float32)
    feat, cnt = _sc_aggregate(xg, ef, zrows, zcol, ones1)
    return _tc_combine(x, feat, cnt, W_self, W_nei,
                       b_self.reshape(1, D_OUT), b_nei.reshape(1, D_OUT))


# speedup vs baseline: 1.0408x; 1.0020x over previous
import functools

import jax
import jax.numpy as jnp
from jax import lax
from jax.experimental import pallas as pl
from jax.experimental.pallas import tpu as pltpu
from jax.experimental.pallas import tpu_sc as plsc

N_NODES = 10000
N_EDGES = 160000
D_IN = 256
D_OUT = 512

H = 128
NC = 2
NS = 16
B = 80
EPT = N_EDGES // NS
NB = EPT // B
AROWS = N_NODES + 8
K = 3
IK = 6
NZT = 16
SLAB = N_NODES // NZT


def _sc_aggregate(xg, ef, zrows, zcol, ones1):
    mesh = plsc.VectorSubcoreMesh(
        core_axis_name="c", subcore_axis_name="s", num_cores=NC,
        num_subcores=NS)

    @functools.partial(
        pl.kernel,
        out_type=(jax.ShapeDtypeStruct((NC * N_NODES, H), jnp.float32),
                  jax.ShapeDtypeStruct((N_NODES, 16), jnp.float32)),
        mesh=mesh,
        scratch_types=[
            pltpu.VMEM_SHARED((AROWS, H), jnp.float32),
            pltpu.VMEM_SHARED((AROWS, 16), jnp.float32),
            pltpu.VMEM((IK, B), jnp.int32),
            pltpu.VMEM((IK, B), jnp.int32),
            pltpu.VMEM((K, B), jnp.int32),
            pltpu.VMEM((K, B, H), jnp.float32),
            pltpu.VMEM((B, 16), jnp.float32),
            pltpu.SemaphoreType.DMA((IK,)),
            pltpu.SemaphoreType.DMA((K,)),
            pltpu.SemaphoreType.DMA((K,)),
            pltpu.SemaphoreType.DMA,
        ],
        compiler_params=pltpu.CompilerParams(use_tc_tiling_on_sc=False),
    )
    def agg(xg_hbm, e_hbm, z_hbm, zc_hbm, ones_hbm, out_hbm, cnt_hbm,
            acc, cac, row_r, col_r, gix_r, feat_r, ones_v,
            isem, gsem, ssem, csem):
        c = lax.axis_index("c")
        s = lax.axis_index("s")

        def start_idx(b, p):
            off = s * EPT + b * B
            pltpu.async_copy(e_hbm.at[pl.ds(off, B)], row_r.at[p],
                             isem.at[p])
            pltpu.async_copy(e_hbm.at[pl.ds(N_EDGES + off, B)],
                             col_r.at[p], isem.at[p])

        def wait_idx(p):
            pltpu.make_async_copy(e_hbm.at[pl.ds(0, B)], row_r.at[p],
                                  isem.at[p]).wait()
            pltpu.make_async_copy(e_hbm.at[pl.ds(0, B)], col_r.at[p],
                                  isem.at[p]).wait()

        def start_gather(p, ip):
            for j in range(B // 16):
                c16 = col_r[ip, pl.ds(j * 16, 16)]
                gix_r[p, pl.ds(j * 16, 16)] = c16 * 2 + c
            pltpu.async_copy(xg_hbm.at[gix_r.at[p]], feat_r.at[p],
                             gsem.at[p])

        def wait_gather(p):
            pltpu.make_async_copy(xg_hbm.at[gix_r.at[p]], feat_r.at[p],
                                  gsem.at[p]).wait()

        def wait_scatter(p):
            pltpu.make_async_copy(feat_r.at[p], acc.at[row_r.at[0]],
                                  ssem.at[p]).wait()

        for p in range(IK):
            start_idx(p, p)
        pltpu.sync_copy(ones_hbm, ones_v)

        @pl.when(s < NZT)
        def _():
            pltpu.sync_copy(z_hbm, acc.at[pl.ds(s * SLAB, SLAB)])

        @pl.when((s < NZT) & (c == 0))
        def _():
            pltpu.sync_copy(zc_hbm, cac.at[pl.ds(s * SLAB, SLAB)])
        plsc.subcore_barrier()
        for t in range(2):
            wait_idx(t)
            start_gather(t, t)

        def body(b, carry):
            p = lax.rem(b, K)
            ip = lax.rem(b, IK)

            @pl.when(b >= 1)
            def _():
                wait_scatter(lax.rem(b - 1, K))

                @pl.when(b - 1 + IK < NB)
                def _():
                    start_idx(b - 1 + IK, lax.rem(b - 1, IK))

            @pl.when(b + 2 < NB)
            def _():
                iq = lax.rem(b + 2, IK)
                wait_idx(iq)
                start_gather(lax.rem(b + 2, K), iq)
            wait_gather(p)
            pltpu.async_copy(feat_r.at[p], acc.at[row_r.at[ip]],
                             ssem.at[p], add=True)

            @pl.when(c == 0)
            def _():
                pltpu.async_copy(ones_v, cac.at[row_r.at[ip]], csem,
                                 add=True)
            return carry

        lax.fori_loop(0, NB, body, 0)
        wait_scatter(lax.rem(NB - 1, K))

        def drain(b, carry):
            pltpu.make_async_copy(ones_v, cac.at[row_r.at[0]], csem).wait()
            return carry

        @pl.when(c == 0)
        def _():
            lax.fori_loop(0, NB, drain, 0)
        plsc.subcore_barrier()

        @pl.when(s < NZT)
        def _():
            pltpu.sync_copy(acc.at[pl.ds(s * SLAB, SLAB)],
                            out_hbm.at[pl.ds(c * N_NODES + s * SLAB, SLAB)])

        @pl.when((s < NZT) & (c == 0))
        def _():
            pltpu.sync_copy(cac.at[pl.ds(s * SLAB, SLAB)],
                            cnt_hbm.at[pl.ds(s * SLAB, SLAB)])

    return agg(xg, ef, zrows, zcol, ones1)


def _tc_combine(x, feat, cnt, W_self, W_nei, b_self2, b_nei2):
    R = 2000
    NR = N_NODES // R

    def body(x_ref, lo_ref, hi_ref, cnt_ref, ws_ref, wn_ref, bs_ref,
             bn_ref, o_ref):
        inv = 1.0 / (cnt_ref[...][:, :1] + 1e-12)
        nlo = lo_ref[...] * inv
        nhi = hi_ref[...] * inv
        xb = x_ref[...]
        ws = ws_ref[...]
        wn = wn_ref[...]
        dn = (((1,), (1,)), ((), ()))
        h = lax.dot_general(xb, ws, dn,
                            preferred_element_type=jnp.float32)
        h = h + lax.dot_general(nlo, wn[:, :H], dn,
                                preferred_element_type=jnp.float32)
        h = h + lax.dot_general(nhi, wn[:, H:], dn,
                                preferred_element_type=jnp.float32)
        h = h + bs_ref[...] + bn_ref[...]
        o_ref[...] = jnp.maximum(h, 0.0)

    return pl.pallas_call(
        body,
        grid=(NR,),
        in_specs=[
            pl.BlockSpec((R, D_IN), lambda i: (i, 0)),
            pl.BlockSpec((R, H), lambda i: (i, 0)),
            pl.BlockSpec((R, H), lambda i: (NR + i, 0)),
            pl.BlockSpec((R, 16), lambda i: (i, 0)),
            pl.BlockSpec((D_OUT, D_IN), lambda i: (0, 0)),
            pl.BlockSpec((D_OUT, D_IN), lambda i: (0, 0)),
            pl.BlockSpec((1, D_OUT), lambda i: (0, 0)),
            pl.BlockSpec((1, D_OUT), lambda i: (0, 0)),
        ],
        out_specs=pl.BlockSpec((R, D_OUT), lambda i: (i, 0)),
        out_shape=jax.ShapeDtypeStruct((N_NODES, D_OUT), jnp.float32),
    )(x, feat, feat, cnt, W_self, W_nei, b_self2, b_nei2)


def kernel(x, edge_index, W_self, b_self, W_nei, b_nei):
    ef = edge_index.astype(jnp.int32).reshape(2 * N_EDGES)
    xg = x.reshape(2 * N_NODES, H)
    zrows = jnp.zeros((SLAB, H), jnp.float32)
    zcol = jnp.zeros((SLAB, 16), jnp.float32)
    ones1 = jnp.ones((B, 16), jnp.float32)
    feat, cnt = _sc_aggregate(xg, ef, zrows, zcol, ones1)
    return _tc_combine(x, feat, cnt, W_self, W_nei,
                       b_self.reshape(1, D_OUT), b_nei.reshape(1, D_OUT))
